# baseline (speedup 1.0000x reference)
import functools

import numpy as np

import jax
import jax.numpy as jnp
from jax import lax
from jax.experimental import pallas as pl
from jax.experimental.pallas import tpu as pltpu
from jax.experimental.pallas import tpu_sc as plsc

N = 10000
E = 320000
D = 128
NCLS = 47

NC = 2
NS = 16
NW = NC * NS
RPW = 320
NPAD = NW * RPW
CHUNK = 128
EPW = 320512
ECH = 2560
NVEC = ECH // 16
NBLK = NVEC // 8
ACC_ROWS = RPW + 8
SENT = RPW + 2
EBLK = 1024
FCH = 256

_mesh = plsc.VectorSubcoreMesh(core_axis_name="c", subcore_axis_name="s")
_params = pltpu.CompilerParams(needs_layout_passes=False)


@functools.partial(
    pl.kernel,
    out_type=[
        jax.ShapeDtypeStruct((NW, EPW), jnp.int32),
        jax.ShapeDtypeStruct((NW, 16), jnp.int32),
    ],
    mesh=_mesh,
    scratch_types=[
        pltpu.VMEM((ECH,), jnp.int32),
        pltpu.VMEM((ECH,), jnp.int32),
        pltpu.VMEM((ECH,), jnp.int32),
        pltpu.VMEM((ECH,), jnp.int32),
        pltpu.VMEM((544,), jnp.int32),
        pltpu.VMEM((16,), jnp.int32),
        pltpu.SemaphoreType.DMA,
        pltpu.SemaphoreType.DMA,
    ],
    compiler_params=_params,
)
def _prep(src_hbm, dst_hbm, edges_out, counts_out,
          srcA, dstA, srcB, dstB, buf, cb, semA, semB):
    wid = lax.axis_index("s") * NC + lax.axis_index("c")
    lo = wid * RPW
    hi = lo + RPW
    iota = lax.iota(jnp.int32, 16)

    def start(j, srcb, dstb, sem):
        pltpu.async_copy(src_hbm.at[pl.ds(j * ECH, ECH)], srcb, sem)
        pltpu.async_copy(dst_hbm.at[pl.ds(j * ECH, ECH)], dstb, sem)

    def wait(srcb, dstb, sem):
        pltpu.make_async_copy(src_hbm.at[pl.ds(0, ECH)], srcb, sem).wait()
        pltpu.make_async_copy(dst_hbm.at[pl.ds(0, ECH)], dstb, sem).wait()

    def scan_chunk(j, carry, srcb, dstb):
        def blk_body(ib, carry):
            c, nfl = carry
            poss, packeds, masks = [], [], []
            for u in range(8):
                off = (ib * 8 + u) * 16
                s = srcb[pl.ds(off, 16)]
                d = dstb[pl.ds(off, 16)]
                m = jnp.logical_and(d >= lo, d < hi)
                poss.append(jnp.cumsum(m.astype(jnp.int32)))
                packeds.append(s * 512 + (d - lo))
                masks.append(m)
            for u in range(8):
                idx = c + poss[u] - 1
                plsc.store_scatter(buf, [idx], packeds[u], mask=masks[u])
                c = c + poss[u][15]
            do = c >= FCH

            @pl.when(do)
            def _():
                pltpu.sync_copy(buf.at[pl.ds(0, FCH)],
                                edges_out.at[wid, pl.ds(nfl * FCH, FCH)])
                for u in range(8):
                    buf[pl.ds(u * 16, 16)] = buf[pl.ds(FCH + u * 16, 16)]

            c = jnp.where(do, c - FCH, c)
            nfl = jnp.where(do, nfl + 1, nfl)
            return c, nfl

        return lax.fori_loop(0, NBLK, blk_body, carry)

    start(0, srcA, dstA, semA)
    NCHE = E // ECH

    def outer(t, carry):
        ja = 2 * t
        jb = 2 * t + 1

        @pl.when(jb < NCHE)
        def _():
            start(jb, srcB, dstB, semB)

        wait(srcA, dstA, semA)
        carry = scan_chunk(ja, carry, srcA, dstA)

        @pl.when(ja + 2 < NCHE)
        def _():
            start(ja + 2, srcA, dstA, semA)

        def do_b(carry):
            wait(srcB, dstB, semB)
            return scan_chunk(jb, carry, srcB, dstB)

        carry = lax.cond(jb < NCHE, do_b, lambda c: c, carry)
        return carry

    c, nfl = lax.fori_loop(0, (NCHE + 1) // 2, outer,
                           (jnp.int32(0), jnp.int32(0)))

    @pl.when(c > 0)
    def _():
        sent = jnp.full((16,), SENT, jnp.int32)
        for j in range(FCH // 16):
            plsc.store_scatter(buf, [c + iota + j * 16], sent)
        pltpu.sync_copy(buf.at[pl.ds(0, FCH)],
                        edges_out.at[wid, pl.ds(nfl * FCH, FCH)])

    nchunks = nfl + jnp.where(c > 0, 1, 0).astype(jnp.int32)
    cb[pl.ds(0, 16)] = jnp.full((16,), nchunks, jnp.int32)
    pltpu.sync_copy(cb, counts_out.at[wid])


@functools.partial(
    pl.kernel,
    out_type=jax.ShapeDtypeStruct((NPAD, D // 2), jnp.int32),
    mesh=_mesh,
    scratch_types=[
        pltpu.VMEM((EBLK,), jnp.int32),
        pltpu.VMEM((FCH,), jnp.int32),
        pltpu.VMEM((FCH,), jnp.int32),
        pltpu.VMEM((FCH, D), jnp.float32),
        pltpu.VMEM((FCH, D), jnp.float32),
        pltpu.VMEM((ACC_ROWS, D // 2), jnp.int32),
        pltpu.VMEM((16,), jnp.int32),
        pltpu.SemaphoreType.DMA,
        pltpu.SemaphoreType.DMA,
    ],
    compiler_params=_params,
)
def _agg(hp_hbm, edges_hbm, counts_hbm, agg_out,
         ebuf, siA, siB, rowsA, rowsB, acc, cb, semA, semB):
    wid = lax.axis_index("s") * NC + lax.axis_index("c")

    def zrow(r, _):
        for k in range(D // 32):
            acc[r, pl.ds(k * 16, 16)] = jnp.zeros((16,), jnp.int32)
        return 0

    lax.fori_loop(0, ACC_ROWS, zrow, 0)

    pltpu.sync_copy(counts_hbm.at[wid], cb)
    nch = cb[pl.ds(0, 16)][0]
    CPB = EBLK // FCH

    def start_gather(q, sib, rows, sem):
        for i in range(FCH // 16):
            v = ebuf[pl.ds(q * FCH + i * 16, 16)]
            sib[pl.ds(i * 16, 16)] = lax.shift_right_logical(v, 9)
        pltpu.async_copy(hp_hbm.at[sib.at[pl.ds(0, CHUNK)]],
                         rows.at[pl.ds(0, CHUNK)], sem)
        pltpu.async_copy(hp_hbm.at[sib.at[pl.ds(CHUNK, CHUNK)]],
                         rows.at[pl.ds(CHUNK, CHUNK)], sem)

    def accum(q, sib, rows, sem):
        pltpu.make_async_copy(hp_hbm.at[sib.at[pl.ds(0, CHUNK)]],
                              rows.at[pl.ds(0, CHUNK)], sem).wait()
        pltpu.make_async_copy(hp_hbm.at[sib.at[pl.ds(CHUNK, CHUNK)]],
                              rows.at[pl.ds(CHUNK, CHUNK)], sem).wait()

        def ebody(i2, _):
            pv = ebuf[pl.ds(q * FCH + i2 * 16, 16)]
            for j in range(16):
                ld = jnp.bitwise_and(pv[j], 511)
                e = i2 * 16 + j
                for k in range(D // 32):
                    r0 = rows[e, pl.ds(k * 32, 16)]
                    r1 = rows[e, pl.ds(k * 32 + 16, 16)]
                    rb = plsc.pack(r0, r1, format=plsc.PackFormat.INTERLEAVED)
                    a = plsc.bitcast(acc[ld, pl.ds(k * 16, 16)], jnp.bfloat16)
                    acc[ld, pl.ds(k * 16, 16)] = plsc.bitcast(
                        jnp.maximum(a, rb), jnp.int32)
            return 0

        lax.fori_loop(0, FCH // 16, ebody, 0)

    def block(b, _):
        base = b * CPB
        pltpu.sync_copy(edges_hbm.at[wid, pl.ds(base * FCH, EBLK)], ebuf)

        @pl.when(base < nch)
        def _():
            start_gather(0, siA, rowsA, semA)

        for q in range(CPB):
            ga = base + q
            sia, rowsa, sema = (siA, rowsA, semA) if q % 2 == 0 else (siB, rowsB, semB)
            sib, rowsb, semb = (siB, rowsB, semB) if q % 2 == 0 else (siA, rowsA, semA)

            if q + 1 < CPB:
                @pl.when(ga + 1 < nch)
                def _():
                    start_gather(q + 1, sib, rowsb, semb)

            @pl.when(ga < nch)
            def _():
                accum(q, sia, rowsa, sema)
        return 0

    nbb = (nch + CPB - 1) // CPB
    lax.fori_loop(0, nbb, block, 0)
    pltpu.sync_copy(acc.at[pl.ds(0, RPW)], agg_out.at[pl.ds(wid * RPW, RPW)])


BR = 1000
GRID = N // BR


def _pool_body(h_ref, w_ref, b_ref, o_ref):
    o_ref[...] = jnp.maximum(
        jnp.dot(h_ref[...], w_ref[...], preferred_element_type=jnp.float32)
        + b_ref[...], 0.0)


def _pool(h, Wp, bp):
    return pl.pallas_call(
        _pool_body,
        grid=(GRID,),
        in_specs=[
            pl.BlockSpec((BR, D), lambda i: (i, 0)),
            pl.BlockSpec((D, D), lambda i: (0, 0)),
            pl.BlockSpec((1, D), lambda i: (0, 0)),
        ],
        out_specs=pl.BlockSpec((BR, D), lambda i: (i, 0)),
        out_shape=jax.ShapeDtypeStruct((N, D), jnp.float32),
    )(h, Wp, bp.reshape(1, D))


def _comb_relu_body(h_ref, a_ref, ws_ref, wn_ref, b_ref, o_ref):
    r = (jnp.dot(h_ref[...], ws_ref[...], preferred_element_type=jnp.float32)
         + jnp.dot(a_ref[...].astype(jnp.float32), wn_ref[...],
                   preferred_element_type=jnp.float32)
         + b_ref[...])
    r = jnp.maximum(r, 0.0)
    n = jnp.sqrt(jnp.sum(r * r, axis=1, keepdims=True))
    o_ref[...] = r / jnp.maximum(n, 1e-12)


def _comb_fused_body(h_ref, a_ref, ws_ref, wn_ref, b_ref, wp_ref, bp_ref,
                     o_ref, op_ref):
    r = (jnp.dot(h_ref[...], ws_ref[...], preferred_element_type=jnp.float32)
         + jnp.dot(a_ref[...].astype(jnp.float32), wn_ref[...],
                   preferred_element_type=jnp.float32)
         + b_ref[...])
    r = jnp.maximum(r, 0.0)
    n = jnp.sqrt(jnp.sum(r * r, axis=1, keepdims=True))
    hn = r / jnp.maximum(n, 1e-12)
    o_ref[...] = hn
    op_ref[...] = jnp.maximum(
        jnp.dot(hn, wp_ref[...], preferred_element_type=jnp.float32)
        + bp_ref[...], 0.0)


def _comb_lsm_body(h_ref, a_ref, ws_ref, wn_ref, b_ref, o_ref):
    r = (jnp.dot(h_ref[...], ws_ref[...], preferred_element_type=jnp.float32)
         + jnp.dot(a_ref[...].astype(jnp.float32), wn_ref[...],
                   preferred_element_type=jnp.float32)
         + b_ref[...])
    r = r - jnp.max(r, axis=1, keepdims=True)
    r = r - jnp.log(jnp.sum(jnp.exp(r), axis=1, keepdims=True))
    n = jnp.sqrt(jnp.sum(r * r, axis=1, keepdims=True))
    o_ref[...] = r / jnp.maximum(n, 1e-12)


def _combine_fused(h, agg, Ws, Wn, b, Wp, bp):
    return pl.pallas_call(
        _comb_fused_body,
        grid=(GRID,),
        in_specs=[
            pl.BlockSpec((BR, D), lambda i: (i, 0)),
            pl.BlockSpec((BR, D), lambda i: (i, 0)),
            pl.BlockSpec((D, D), lambda i: (0, 0)),
            pl.BlockSpec((D, D), lambda i: (0, 0)),
            pl.BlockSpec((1, D), lambda i: (0, 0)),
            pl.BlockSpec((D, D), lambda i: (0, 0)),
            pl.BlockSpec((1, D), lambda i: (0, 0)),
        ],
        out_specs=[
            pl.BlockSpec((BR, D), lambda i: (i, 0)),
            pl.BlockSpec((BR, D), lambda i: (i, 0)),
        ],
        out_shape=[
            jax.ShapeDtypeStruct((N, D), jnp.float32),
            jax.ShapeDtypeStruct((N, D), jnp.float32),
        ],
    )(h, agg, Ws, Wn, b.reshape(1, D), Wp, bp.reshape(1, D))


def _combine_last(h, agg, Ws, Wn, b):
    dout = Ws.shape[1]
    return pl.pallas_call(
        _comb_lsm_body,
        grid=(GRID,),
        in_specs=[
            pl.BlockSpec((BR, D), lambda i: (i, 0)),
            pl.BlockSpec((BR, D), lambda i: (i, 0)),
            pl.BlockSpec((D, dout), lambda i: (0, 0)),
            pl.BlockSpec((D, dout), lambda i: (0, 0)),
            pl.BlockSpec((1, dout), lambda i: (0, 0)),
        ],
        out_specs=pl.BlockSpec((BR, dout), lambda i: (i, 0)),
        out_shape=jax.ShapeDtypeStruct((N, dout), jnp.float32),
    )(h, agg, Ws, Wn, b.reshape(1, dout))


_PERM = np.arange(D).reshape(D // 32, 2, 16).transpose(0, 2, 1).reshape(D)


def _raw_view(agg32):
    return lax.bitcast_convert_type(agg32, jnp.bfloat16).reshape(NPAD, D)


def kernel(x, edge_index, Wp0, bp0, Ws0, Wn0, b0, Wp1, bp1, Ws1, Wn1, b1,
           Wp2, bp2, Ws2, Wn2, b2):
    src = edge_index[0]
    dst = edge_index[1]
    edges, counts = _prep(src, dst)
    Wn0p, Wn1p, Wn2p = Wn0[_PERM], Wn1[_PERM], Wn2[_PERM]
    hp = _pool(x, Wp0, bp0)
    a0 = _raw_view(_agg(hp, edges, counts))
    h1, hp1 = _combine_fused(x, a0, Ws0, Wn0p, b0, Wp1, bp1)
    a1 = _raw_view(_agg(hp1, edges, counts))
    h2, hp2 = _combine_fused(h1, a1, Ws1, Wn1p, b1, Wp2, bp2)
    a2 = _raw_view(_agg(hp2, edges, counts))
    return _combine_last(h2, a2, Ws2, Wn2p, b2)


# speedup vs baseline: 1.1572x; 1.1572x over previous
import functools

import numpy as np

import jax
import jax.numpy as jnp
from jax import lax
from jax.experimental import pallas as pl
from jax.experimental.pallas import tpu as pltpu
from jax.experimental.pallas import tpu_sc as plsc

N = 10000
E = 320000
D = 128
NCLS = 47

NC = 2
NS = 16
NW = NC * NS
RPW = 320
NPAD = NW * RPW
CHUNK = 128
EPW = 320512
ECH = 2560
NVEC = ECH // 16
NBLK = NVEC // 8
ACC_ROWS = RPW + 8
SENT = RPW + 2
EBLK = 1024
FCH = 256

_mesh = plsc.VectorSubcoreMesh(core_axis_name="c", subcore_axis_name="s")
_params = pltpu.CompilerParams(needs_layout_passes=False)


@functools.partial(
    pl.kernel,
    out_type=[
        jax.ShapeDtypeStruct((NW, EPW), jnp.int32),
        jax.ShapeDtypeStruct((NW, 16), jnp.int32),
    ],
    mesh=_mesh,
    scratch_types=[
        pltpu.VMEM((ECH,), jnp.int32),
        pltpu.VMEM((ECH,), jnp.int32),
        pltpu.VMEM((ECH,), jnp.int32),
        pltpu.VMEM((ECH,), jnp.int32),
        pltpu.VMEM((272,), jnp.int32),
        pltpu.VMEM((16,), jnp.int32),
        pltpu.SemaphoreType.DMA,
        pltpu.SemaphoreType.DMA,
    ],
    compiler_params=_params,
)
def _prep(src_hbm, dst_hbm, edges_out, counts_out,
          srcA, dstA, srcB, dstB, buf, cb, semA, semB):
    wid = lax.axis_index("s") * NC + lax.axis_index("c")
    lo = wid * RPW
    hi = lo + RPW
    iota = lax.iota(jnp.int32, 16)

    def start(j, srcb, dstb, sem):
        pltpu.async_copy(src_hbm.at[pl.ds(j * ECH, ECH)], srcb, sem)
        pltpu.async_copy(dst_hbm.at[pl.ds(j * ECH, ECH)], dstb, sem)

    def wait(srcb, dstb, sem):
        pltpu.make_async_copy(src_hbm.at[pl.ds(0, ECH)], srcb, sem).wait()
        pltpu.make_async_copy(dst_hbm.at[pl.ds(0, ECH)], dstb, sem).wait()

    def scan_chunk(j, carry, srcb, dstb):
        def blk_body(ib, carry):
            c, nfl = carry
            poss, packeds, masks = [], [], []
            for u in range(8):
                off = (ib * 8 + u) * 16
                s = srcb[pl.ds(off, 16)]
                d = dstb[pl.ds(off, 16)]
                m = jnp.logical_and(d >= lo, d < hi)
                poss.append(jnp.cumsum(m.astype(jnp.int32)))
                packeds.append(s * 512 + (d - lo))
                masks.append(m)
            for u in range(8):
                idx = c + poss[u] - 1
                plsc.store_scatter(buf, [idx], packeds[u], mask=masks[u])
                c = c + poss[u][15]
            do = c >= CHUNK

            @pl.when(do)
            def _():
                pltpu.sync_copy(buf.at[pl.ds(0, CHUNK)],
                                edges_out.at[wid, pl.ds(nfl * CHUNK, CHUNK)])
                for u in range(8):
                    buf[pl.ds(u * 16, 16)] = buf[pl.ds(CHUNK + u * 16, 16)]

            c = jnp.where(do, c - CHUNK, c)
            nfl = jnp.where(do, nfl + 1, nfl)
            return c, nfl

        return lax.fori_loop(0, NBLK, blk_body, carry)

    start(0, srcA, dstA, semA)
    NCHE = E // ECH

    def outer(t, carry):
        ja = 2 * t
        jb = 2 * t + 1

        @pl.when(jb < NCHE)
        def _():
            start(jb, srcB, dstB, semB)

        wait(srcA, dstA, semA)
        carry = scan_chunk(ja, carry, srcA, dstA)

        @pl.when(ja + 2 < NCHE)
        def _():
            start(ja + 2, srcA, dstA, semA)

        def do_b(carry):
            wait(srcB, dstB, semB)
            return scan_chunk(jb, carry, srcB, dstB)

        carry = lax.cond(jb < NCHE, do_b, lambda c: c, carry)
        return carry

    c, nfl = lax.fori_loop(0, (NCHE + 1) // 2, outer,
                           (jnp.int32(0), jnp.int32(0)))

    @pl.when(c > 0)
    def _():
        sent = jnp.full((16,), SENT, jnp.int32)
        for j in range(CHUNK // 16):
            plsc.store_scatter(buf, [c + iota + j * 16], sent)
        pltpu.sync_copy(buf.at[pl.ds(0, CHUNK)],
                        edges_out.at[wid, pl.ds(nfl * CHUNK, CHUNK)])

    nchunks = nfl + jnp.where(c > 0, 1, 0).astype(jnp.int32)
    cb[pl.ds(0, 16)] = jnp.full((16,), nchunks, jnp.int32)
    pltpu.sync_copy(cb, counts_out.at[wid])


@functools.partial(
    pl.kernel,
    out_type=jax.ShapeDtypeStruct((NPAD, D // 2), jnp.int32),
    mesh=_mesh,
    scratch_types=[
        pltpu.VMEM((EBLK,), jnp.int32),
        pltpu.VMEM((CHUNK,), jnp.int32),
        pltpu.VMEM((CHUNK,), jnp.int32),
        pltpu.VMEM((CHUNK, D), jnp.float32),
        pltpu.VMEM((CHUNK, D), jnp.float32),
        pltpu.VMEM((ACC_ROWS, D // 2), jnp.int32),
        pltpu.VMEM((16,), jnp.int32),
        pltpu.SemaphoreType.DMA,
        pltpu.SemaphoreType.DMA,
    ],
    compiler_params=_params,
)
def _agg(hp_hbm, edges_hbm, counts_hbm, agg_out,
         ebuf, siA, siB, rowsA, rowsB, acc, cb, semA, semB):
    wid = lax.axis_index("s") * NC + lax.axis_index("c")

    def zrow(r, _):
        for k in range(D // 32):
            acc[r, pl.ds(k * 16, 16)] = jnp.zeros((16,), jnp.int32)
        return 0

    lax.fori_loop(0, ACC_ROWS, zrow, 0)

    pltpu.sync_copy(counts_hbm.at[wid], cb)
    nch = cb[pl.ds(0, 16)][0]
    CPB = EBLK // CHUNK

    def start_gather(q, sib, rows, sem):
        for i in range(CHUNK // 16):
            v = ebuf[pl.ds(q * CHUNK + i * 16, 16)]
            sib[pl.ds(i * 16, 16)] = lax.shift_right_logical(v, 9)
        pltpu.async_copy(hp_hbm.at[sib], rows, sem)

    def accum(q, sib, rows, sem):
        pltpu.make_async_copy(hp_hbm.at[sib], rows, sem).wait()

        def ebody(i2, _):
            pv = ebuf[pl.ds(q * CHUNK + i2 * 16, 16)]
            for j in range(16):
                ld = jnp.bitwise_and(pv[j], 511)
                e = i2 * 16 + j
                for k in range(D // 32):
                    r0 = rows[e, pl.ds(k * 32, 16)]
                    r1 = rows[e, pl.ds(k * 32 + 16, 16)]
                    rb = plsc.pack(r0, r1, format=plsc.PackFormat.INTERLEAVED)
                    a = plsc.bitcast(acc[ld, pl.ds(k * 16, 16)], jnp.bfloat16)
                    acc[ld, pl.ds(k * 16, 16)] = plsc.bitcast(
                        jnp.maximum(a, rb), jnp.int32)
            return 0

        lax.fori_loop(0, CHUNK // 16, ebody, 0)

    def block(b, _):
        base = b * CPB
        pltpu.sync_copy(edges_hbm.at[wid, pl.ds(base * CHUNK, EBLK)], ebuf)

        @pl.when(base < nch)
        def _():
            start_gather(0, siA, rowsA, semA)

        for q in range(CPB):
            ga = base + q
            sia, rowsa, sema = (siA, rowsA, semA) if q % 2 == 0 else (siB, rowsB, semB)
            sib, rowsb, semb = (siB, rowsB, semB) if q % 2 == 0 else (siA, rowsA, semA)

            if q + 1 < CPB:
                @pl.when(ga + 1 < nch)
                def _():
                    start_gather(q + 1, sib, rowsb, semb)

            @pl.when(ga < nch)
            def _():
                accum(q, sia, rowsa, sema)
        return 0

    nbb = (nch + CPB - 1) // CPB
    lax.fori_loop(0, nbb, block, 0)
    pltpu.sync_copy(acc.at[pl.ds(0, RPW)], agg_out.at[pl.ds(wid * RPW, RPW)])


BR = 1000
GRID = N // BR


def _pool_body(h_ref, w_ref, b_ref, o_ref):
    o_ref[...] = jnp.maximum(
        jnp.dot(h_ref[...], w_ref[...], preferred_element_type=jnp.float32)
        + b_ref[...], 0.0)


def _pool(h, Wp, bp):
    return pl.pallas_call(
        _pool_body,
        grid=(GRID,),
        in_specs=[
            pl.BlockSpec((BR, D), lambda i: (i, 0)),
            pl.BlockSpec((D, D), lambda i: (0, 0)),
            pl.BlockSpec((1, D), lambda i: (0, 0)),
        ],
        out_specs=pl.BlockSpec((BR, D), lambda i: (i, 0)),
        out_shape=jax.ShapeDtypeStruct((N, D), jnp.float32),
    )(h, Wp, bp.reshape(1, D))


def _comb_relu_body(h_ref, a_ref, ws_ref, wn_ref, b_ref, o_ref):
    r = (jnp.dot(h_ref[...], ws_ref[...], preferred_element_type=jnp.float32)
         + jnp.dot(a_ref[...].astype(jnp.float32), wn_ref[...],
                   preferred_element_type=jnp.float32)
         + b_ref[...])
    r = jnp.maximum(r, 0.0)
    n = jnp.sqrt(jnp.sum(r * r, axis=1, keepdims=True))
    o_ref[...] = r / jnp.maximum(n, 1e-12)


def _comb_fused_body(h_ref, a_ref, ws_ref, wn_ref, b_ref, wp_ref, bp_ref,
                     o_ref, op_ref):
    r = (jnp.dot(h_ref[...], ws_ref[...], preferred_element_type=jnp.float32)
         + jnp.dot(a_ref[...].astype(jnp.float32), wn_ref[...],
                   preferred_element_type=jnp.float32)
         + b_ref[...])
    r = jnp.maximum(r, 0.0)
    n = jnp.sqrt(jnp.sum(r * r, axis=1, keepdims=True))
    hn = r / jnp.maximum(n, 1e-12)
    o_ref[...] = hn
    op_ref[...] = jnp.maximum(
        jnp.dot(hn, wp_ref[...], preferred_element_type=jnp.float32)
        + bp_ref[...], 0.0)


def _comb_lsm_body(h_ref, a_ref, ws_ref, wn_ref, b_ref, o_ref):
    r = (jnp.dot(h_ref[...], ws_ref[...], preferred_element_type=jnp.float32)
         + jnp.dot(a_ref[...].astype(jnp.float32), wn_ref[...],
                   preferred_element_type=jnp.float32)
         + b_ref[...])
    r = r - jnp.max(r, axis=1, keepdims=True)
    r = r - jnp.log(jnp.sum(jnp.exp(r), axis=1, keepdims=True))
    n = jnp.sqrt(jnp.sum(r * r, axis=1, keepdims=True))
    o_ref[...] = r / jnp.maximum(n, 1e-12)


def _combine_fused(h, agg, Ws, Wn, b, Wp, bp):
    return pl.pallas_call(
        _comb_fused_body,
        grid=(GRID,),
        in_specs=[
            pl.BlockSpec((BR, D), lambda i: (i, 0)),
            pl.BlockSpec((BR, D), lambda i: (i, 0)),
            pl.BlockSpec((D, D), lambda i: (0, 0)),
            pl.BlockSpec((D, D), lambda i: (0, 0)),
            pl.BlockSpec((1, D), lambda i: (0, 0)),
            pl.BlockSpec((D, D), lambda i: (0, 0)),
            pl.BlockSpec((1, D), lambda i: (0, 0)),
        ],
        out_specs=[
            pl.BlockSpec((BR, D), lambda i: (i, 0)),
            pl.BlockSpec((BR, D), lambda i: (i, 0)),
        ],
        out_shape=[
            jax.ShapeDtypeStruct((N, D), jnp.float32),
            jax.ShapeDtypeStruct((N, D), jnp.float32),
        ],
    )(h, agg, Ws, Wn, b.reshape(1, D), Wp, bp.reshape(1, D))


def _combine_last(h, agg, Ws, Wn, b):
    dout = Ws.shape[1]
    return pl.pallas_call(
        _comb_lsm_body,
        grid=(GRID,),
        in_specs=[
            pl.BlockSpec((BR, D), lambda i: (i, 0)),
            pl.BlockSpec((BR, D), lambda i: (i, 0)),
            pl.BlockSpec((D, dout), lambda i: (0, 0)),
            pl.BlockSpec((D, dout), lambda i: (0, 0)),
            pl.BlockSpec((1, dout), lambda i: (0, 0)),
        ],
        out_specs=pl.BlockSpec((BR, dout), lambda i: (i, 0)),
        out_shape=jax.ShapeDtypeStruct((N, dout), jnp.float32),
    )(h, agg, Ws, Wn, b.reshape(1, dout))


_PERM = np.arange(D).reshape(D // 32, 2, 16).transpose(0, 2, 1).reshape(D)


def _raw_view(agg32):
    return lax.bitcast_convert_type(agg32, jnp.bfloat16).reshape(NPAD, D)


def kernel(x, edge_index, Wp0, bp0, Ws0, Wn0, b0, Wp1, bp1, Ws1, Wn1, b1,
           Wp2, bp2, Ws2, Wn2, b2):
    src = edge_index[0]
    dst = edge_index[1]
    edges, counts = _prep(src, dst)
    Wn0p, Wn1p, Wn2p = Wn0[_PERM], Wn1[_PERM], Wn2[_PERM]
    hp = _pool(x, Wp0, bp0)
    a0 = _raw_view(_agg(hp, edges, counts))
    h1, hp1 = _combine_fused(x, a0, Ws0, Wn0p, b0, Wp1, bp1)
    a1 = _raw_view(_agg(hp1, edges, counts))
    h2, hp2 = _combine_fused(h1, a1, Ws1, Wn1p, b1, Wp2, bp2)
    a2 = _raw_view(_agg(hp2, edges, counts))
    return _combine_last(h2, a2, Ws2, Wn2p, b2)


# speedup vs baseline: 1.1741x; 1.0146x over previous
import functools

import numpy as np

import jax
import jax.numpy as jnp
from jax import lax
from jax.experimental import pallas as pl
from jax.experimental.pallas import tpu as pltpu
from jax.experimental.pallas import tpu_sc as plsc

N = 10000
E = 320000
D = 128
NCLS = 47

NC = 2
NS = 16
NW = NC * NS
RPW = 320
NPAD = NW * RPW
CHUNK = 128
EPW = 321536
ECH = 2560
NVEC = ECH // 16
NBLK = NVEC // 8
ACC_ROWS = RPW + 8
SENT = RPW + 2
EBLK = 2048
FCH = 256

_mesh = plsc.VectorSubcoreMesh(core_axis_name="c", subcore_axis_name="s")
_params = pltpu.CompilerParams(needs_layout_passes=False)


@functools.partial(
    pl.kernel,
    out_type=[
        jax.ShapeDtypeStruct((NW, EPW), jnp.int32),
        jax.ShapeDtypeStruct((NW, 16), jnp.int32),
    ],
    mesh=_mesh,
    scratch_types=[
        pltpu.VMEM((ECH,), jnp.int32),
        pltpu.VMEM((ECH,), jnp.int32),
        pltpu.VMEM((ECH,), jnp.int32),
        pltpu.VMEM((ECH,), jnp.int32),
        pltpu.VMEM((272,), jnp.int32),
        pltpu.VMEM((16,), jnp.int32),
        pltpu.SemaphoreType.DMA,
        pltpu.SemaphoreType.DMA,
    ],
    compiler_params=_params,
)
def _prep(src_hbm, dst_hbm, edges_out, counts_out,
          srcA, dstA, srcB, dstB, buf, cb, semA, semB):
    wid = lax.axis_index("s") * NC + lax.axis_index("c")
    lo = wid * RPW
    hi = lo + RPW
    iota = lax.iota(jnp.int32, 16)

    def start(j, srcb, dstb, sem):
        pltpu.async_copy(src_hbm.at[pl.ds(j * ECH, ECH)], srcb, sem)
        pltpu.async_copy(dst_hbm.at[pl.ds(j * ECH, ECH)], dstb, sem)

    def wait(srcb, dstb, sem):
        pltpu.make_async_copy(src_hbm.at[pl.ds(0, ECH)], srcb, sem).wait()
        pltpu.make_async_copy(dst_hbm.at[pl.ds(0, ECH)], dstb, sem).wait()

    def scan_chunk(j, carry, srcb, dstb):
        def blk_body(ib, carry):
            c, nfl = carry
            poss, packeds, masks = [], [], []
            for u in range(8):
                off = (ib * 8 + u) * 16
                s = srcb[pl.ds(off, 16)]
                d = dstb[pl.ds(off, 16)]
                m = jnp.logical_and(d >= lo, d < hi)
                poss.append(jnp.cumsum(m.astype(jnp.int32)))
                packeds.append(s * 512 + (d - lo))
                masks.append(m)
            for u in range(8):
                idx = c + poss[u] - 1
                plsc.store_scatter(buf, [idx], packeds[u], mask=masks[u])
                c = c + poss[u][15]
            do = c >= CHUNK

            @pl.when(do)
            def _():
                pltpu.sync_copy(buf.at[pl.ds(0, CHUNK)],
                                edges_out.at[wid, pl.ds(nfl * CHUNK, CHUNK)])
                for u in range(8):
                    buf[pl.ds(u * 16, 16)] = buf[pl.ds(CHUNK + u * 16, 16)]

            c = jnp.where(do, c - CHUNK, c)
            nfl = jnp.where(do, nfl + 1, nfl)
            return c, nfl

        return lax.fori_loop(0, NBLK, blk_body, carry)

    start(0, srcA, dstA, semA)
    NCHE = E // ECH

    def outer(t, carry):
        ja = 2 * t
        jb = 2 * t + 1

        @pl.when(jb < NCHE)
        def _():
            start(jb, srcB, dstB, semB)

        wait(srcA, dstA, semA)
        carry = scan_chunk(ja, carry, srcA, dstA)

        @pl.when(ja + 2 < NCHE)
        def _():
            start(ja + 2, srcA, dstA, semA)

        def do_b(carry):
            wait(srcB, dstB, semB)
            return scan_chunk(jb, carry, srcB, dstB)

        carry = lax.cond(jb < NCHE, do_b, lambda c: c, carry)
        return carry

    c, nfl = lax.fori_loop(0, (NCHE + 1) // 2, outer,
                           (jnp.int32(0), jnp.int32(0)))

    @pl.when(c > 0)
    def _():
        sent = jnp.full((16,), SENT, jnp.int32)
        for j in range(CHUNK // 16):
            plsc.store_scatter(buf, [c + iota + j * 16], sent)
        pltpu.sync_copy(buf.at[pl.ds(0, CHUNK)],
                        edges_out.at[wid, pl.ds(nfl * CHUNK, CHUNK)])

    nchunks = nfl + jnp.where(c > 0, 1, 0).astype(jnp.int32)
    cb[pl.ds(0, 16)] = jnp.full((16,), nchunks, jnp.int32)
    pltpu.sync_copy(cb, counts_out.at[wid])


@functools.partial(
    pl.kernel,
    out_type=jax.ShapeDtypeStruct((NPAD, D // 2), jnp.int32),
    mesh=_mesh,
    scratch_types=[
        pltpu.VMEM((EBLK,), jnp.int32),
        pltpu.VMEM((CHUNK,), jnp.int32),
        pltpu.VMEM((CHUNK,), jnp.int32),
        pltpu.VMEM((CHUNK, D), jnp.float32),
        pltpu.VMEM((CHUNK, D), jnp.float32),
        pltpu.VMEM((ACC_ROWS, D // 2), jnp.int32),
        pltpu.VMEM((16,), jnp.int32),
        pltpu.SemaphoreType.DMA,
        pltpu.SemaphoreType.DMA,
    ],
    compiler_params=_params,
)
def _agg(hp_hbm, edges_hbm, counts_hbm, agg_out,
         ebuf, siA, siB, rowsA, rowsB, acc, cb, semA, semB):
    wid = lax.axis_index("s") * NC + lax.axis_index("c")

    def zrow(r, _):
        for k in range(D // 32):
            acc[r, pl.ds(k * 16, 16)] = jnp.zeros((16,), jnp.int32)
        return 0

    lax.fori_loop(0, ACC_ROWS, zrow, 0)

    pltpu.sync_copy(counts_hbm.at[wid], cb)
    nch = cb[pl.ds(0, 16)][0]
    CPB = EBLK // CHUNK

    def start_gather(q, sib, rows, sem):
        for i in range(CHUNK // 16):
            v = ebuf[pl.ds(q * CHUNK + i * 16, 16)]
            sib[pl.ds(i * 16, 16)] = lax.shift_right_logical(v, 9)
        pltpu.async_copy(hp_hbm.at[sib], rows, sem)

    def accum(q, sib, rows, sem):
        pltpu.make_async_copy(hp_hbm.at[sib], rows, sem).wait()

        def ebody(i2, _):
            pv = ebuf[pl.ds(q * CHUNK + i2 * 16, 16)]
            for j in range(16):
                ld = jnp.bitwise_and(pv[j], 511)
                e = i2 * 16 + j
                for k in range(D // 32):
                    r0 = rows[e, pl.ds(k * 32, 16)]
                    r1 = rows[e, pl.ds(k * 32 + 16, 16)]
                    rb = plsc.pack(r0, r1, format=plsc.PackFormat.INTERLEAVED)
                    a = plsc.bitcast(acc[ld, pl.ds(k * 16, 16)], jnp.bfloat16)
                    acc[ld, pl.ds(k * 16, 16)] = plsc.bitcast(
                        jnp.maximum(a, rb), jnp.int32)
            return 0

        lax.fori_loop(0, CHUNK // 16, ebody, 0)

    def block(b, _):
        base = b * CPB
        pltpu.sync_copy(edges_hbm.at[wid, pl.ds(base * CHUNK, EBLK)], ebuf)

        @pl.when(base < nch)
        def _():
            start_gather(0, siA, rowsA, semA)

        for q in range(CPB):
            ga = base + q
            sia, rowsa, sema = (siA, rowsA, semA) if q % 2 == 0 else (siB, rowsB, semB)
            sib, rowsb, semb = (siB, rowsB, semB) if q % 2 == 0 else (siA, rowsA, semA)

            if q + 1 < CPB:
                @pl.when(ga + 1 < nch)
                def _():
                    start_gather(q + 1, sib, rowsb, semb)

            @pl.when(ga < nch)
            def _():
                accum(q, sia, rowsa, sema)
        return 0

    nbb = (nch + CPB - 1) // CPB
    lax.fori_loop(0, nbb, block, 0)
    pltpu.sync_copy(acc.at[pl.ds(0, RPW)], agg_out.at[pl.ds(wid * RPW, RPW)])


BR = 1000
GRID = N // BR


def _pool_body(h_ref, w_ref, b_ref, o_ref):
    o_ref[...] = jnp.maximum(
        jnp.dot(h_ref[...], w_ref[...], preferred_element_type=jnp.float32)
        + b_ref[...], 0.0)


def _pool(h, Wp, bp):
    return pl.pallas_call(
        _pool_body,
        grid=(GRID,),
        in_specs=[
            pl.BlockSpec((BR, D), lambda i: (i, 0)),
            pl.BlockSpec((D, D), lambda i: (0, 0)),
            pl.BlockSpec((1, D), lambda i: (0, 0)),
        ],
        out_specs=pl.BlockSpec((BR, D), lambda i: (i, 0)),
        out_shape=jax.ShapeDtypeStruct((N, D), jnp.float32),
    )(h, Wp, bp.reshape(1, D))


def _comb_relu_body(h_ref, a_ref, ws_ref, wn_ref, b_ref, o_ref):
    r = (jnp.dot(h_ref[...], ws_ref[...], preferred_element_type=jnp.float32)
         + jnp.dot(a_ref[...].astype(jnp.float32), wn_ref[...],
                   preferred_element_type=jnp.float32)
         + b_ref[...])
    r = jnp.maximum(r, 0.0)
    n = jnp.sqrt(jnp.sum(r * r, axis=1, keepdims=True))
    o_ref[...] = r / jnp.maximum(n, 1e-12)


def _comb_fused_body(h_ref, a_ref, ws_ref, wn_ref, b_ref, wp_ref, bp_ref,
                     o_ref, op_ref):
    r = (jnp.dot(h_ref[...], ws_ref[...], preferred_element_type=jnp.float32)
         + jnp.dot(a_ref[...].astype(jnp.float32), wn_ref[...],
                   preferred_element_type=jnp.float32)
         + b_ref[...])
    r = jnp.maximum(r, 0.0)
    n = jnp.sqrt(jnp.sum(r * r, axis=1, keepdims=True))
    hn = r / jnp.maximum(n, 1e-12)
    o_ref[...] = hn
    op_ref[...] = jnp.maximum(
        jnp.dot(hn, wp_ref[...], preferred_element_type=jnp.float32)
        + bp_ref[...], 0.0)


def _comb_lsm_body(h_ref, a_ref, ws_ref, wn_ref, b_ref, o_ref):
    r = (jnp.dot(h_ref[...], ws_ref[...], preferred_element_type=jnp.float32)
         + jnp.dot(a_ref[...].astype(jnp.float32), wn_ref[...],
                   preferred_element_type=jnp.float32)
         + b_ref[...])
    r = r - jnp.max(r, axis=1, keepdims=True)
    r = r - jnp.log(jnp.sum(jnp.exp(r), axis=1, keepdims=True))
    n = jnp.sqrt(jnp.sum(r * r, axis=1, keepdims=True))
    o_ref[...] = r / jnp.maximum(n, 1e-12)


def _combine_fused(h, agg, Ws, Wn, b, Wp, bp):
    return pl.pallas_call(
        _comb_fused_body,
        grid=(GRID,),
        in_specs=[
            pl.BlockSpec((BR, D), lambda i: (i, 0)),
            pl.BlockSpec((BR, D), lambda i: (i, 0)),
            pl.BlockSpec((D, D), lambda i: (0, 0)),
            pl.BlockSpec((D, D), lambda i: (0, 0)),
            pl.BlockSpec((1, D), lambda i: (0, 0)),
            pl.BlockSpec((D, D), lambda i: (0, 0)),
            pl.BlockSpec((1, D), lambda i: (0, 0)),
        ],
        out_specs=[
            pl.BlockSpec((BR, D), lambda i: (i, 0)),
            pl.BlockSpec((BR, D), lambda i: (i, 0)),
        ],
        out_shape=[
            jax.ShapeDtypeStruct((N, D), jnp.float32),
            jax.ShapeDtypeStruct((N, D), jnp.float32),
        ],
    )(h, agg, Ws, Wn, b.reshape(1, D), Wp, bp.reshape(1, D))


def _combine_last(h, agg, Ws, Wn, b):
    dout = Ws.shape[1]
    return pl.pallas_call(
        _comb_lsm_body,
        grid=(GRID,),
        in_specs=[
            pl.BlockSpec((BR, D), lambda i: (i, 0)),
            pl.BlockSpec((BR, D), lambda i: (i, 0)),
            pl.BlockSpec((D, dout), lambda i: (0, 0)),
            pl.BlockSpec((D, dout), lambda i: (0, 0)),
            pl.BlockSpec((1, dout), lambda i: (0, 0)),
        ],
        out_specs=pl.BlockSpec((BR, dout), lambda i: (i, 0)),
        out_shape=jax.ShapeDtypeStruct((N, dout), jnp.float32),
    )(h, agg, Ws, Wn, b.reshape(1, dout))


_PERM = np.arange(D).reshape(D // 32, 2, 16).transpose(0, 2, 1).reshape(D)


def _raw_view(agg32):
    return lax.bitcast_convert_type(agg32, jnp.bfloat16).reshape(NPAD, D)


def kernel(x, edge_index, Wp0, bp0, Ws0, Wn0, b0, Wp1, bp1, Ws1, Wn1, b1,
           Wp2, bp2, Ws2, Wn2, b2):
    src = edge_index[0]
    dst = edge_index[1]
    edges, counts = _prep(src, dst)
    Wn0p, Wn1p, Wn2p = Wn0[_PERM], Wn1[_PERM], Wn2[_PERM]
    hp = _pool(x, Wp0, bp0)
    a0 = _raw_view(_agg(hp, edges, counts))
    h1, hp1 = _combine_fused(x, a0, Ws0, Wn0p, b0, Wp1, bp1)
    a1 = _raw_view(_agg(hp1, edges, counts))
    h2, hp2 = _combine_fused(h1, a1, Ws1, Wn1p, b1, Wp2, bp2)
    a2 = _raw_view(_agg(hp2, edges, counts))
    return _combine_last(h2, a2, Ws2, Wn2p, b2)
